# Initial kernel scaffold; baseline (speedup 1.0000x reference)
import jax
import jax.numpy as jnp
from jax.experimental import pallas as pl


def kernel(x, edge_index, batch, params):
    raise NotImplementedError("write your pallas kernel here")


# speedup vs baseline: 8.4064x; 8.4064x over previous
import functools

import jax
import jax.numpy as jnp
from jax import lax
from jax.experimental import pallas as pl
from jax.experimental.pallas import tpu as pltpu
from jax.experimental.pallas import tpu_sc as plsc

N = 10000
E = 320000
D = 128
G = 16

NC = 2
NS = 16
DH = D // NC
EPS_ = E // NS
CH = 80
NCHUNK = EPS_ // CH
RPS = N // NS
ZR = 125

BLK = 1000
NB = N // BLK


def _sc_body(hr, srcr, dstr, out, sidx, didx, rows, zbuf, agg, sem0, sem1):
    cid = lax.axis_index("c")
    sid = lax.axis_index("s")

    pltpu.sync_copy(srcr.at[sid], sidx)
    pltpu.sync_copy(dstr.at[sid], didx)

    zv = jnp.zeros((16,), jnp.float32)

    def zrow(i, c):
        def zcol(k, c2):
            zbuf[i, pl.ds(k * 16, 16)] = zv
            return c2
        return lax.fori_loop(0, DH // 16, zcol, c)

    lax.fori_loop(0, ZR, zrow, 0)
    for r in range(RPS // ZR):
        pltpu.sync_copy(zbuf, agg.at[pl.ds(sid * RPS + r * ZR, ZR)])
    plsc.subcore_barrier()

    table = hr.at[cid]
    pltpu.async_copy(table.at[sidx.at[0]], rows.at[0], sem0)

    def chunk2(jj, c):
        j0 = 2 * jj

        @pl.when(j0 + 1 < NCHUNK)
        def _():
            pltpu.async_copy(table.at[sidx.at[j0 + 1]], rows.at[1], sem1)
        pltpu.make_async_copy(table.at[sidx.at[j0]], rows.at[0], sem0).wait()
        pltpu.sync_copy(rows.at[0], agg.at[didx.at[j0]], add=True)

        @pl.when(j0 + 2 < NCHUNK)
        def _():
            pltpu.async_copy(table.at[sidx.at[j0 + 2]], rows.at[0], sem0)

        @pl.when(j0 + 1 < NCHUNK)
        def _():
            pltpu.make_async_copy(table.at[sidx.at[j0 + 1]], rows.at[1],
                                  sem1).wait()
            pltpu.sync_copy(rows.at[1], agg.at[didx.at[j0 + 1]], add=True)
        return c

    lax.fori_loop(0, (NCHUNK + 1) // 2, chunk2, 0)
    plsc.subcore_barrier()

    start = pl.multiple_of(sid * 624, 16)

    @pl.when(sid < NS - 1)
    def _():
        pltpu.sync_copy(agg.at[pl.ds(start, 624)],
                        out.at[cid, pl.ds(start, 624)])

    @pl.when(sid == NS - 1)
    def _():
        pltpu.sync_copy(agg.at[pl.ds(start, 640)],
                        out.at[cid, pl.ds(start, 640)])


_sc_aggregate = pl.kernel(
    _sc_body,
    out_type=jax.ShapeDtypeStruct((NC, N, DH), jnp.float32),
    mesh=plsc.VectorSubcoreMesh(core_axis_name="c", subcore_axis_name="s"),
    scratch_types=[
        pltpu.VMEM((NCHUNK, CH), jnp.int32),
        pltpu.VMEM((NCHUNK, CH), jnp.int32),
        pltpu.VMEM((2, CH, DH), jnp.float32),
        pltpu.VMEM((ZR, DH), jnp.float32),
        pltpu.VMEM_SHARED((N, DH), jnp.float32),
        pltpu.SemaphoreType.DMA,
        pltpu.SemaphoreType.DMA,
    ],
    compiler_params=pltpu.CompilerParams(use_tc_tiling_on_sc=False),
)


def _proj_body(x_ref, w_ref, b_ref, h_ref, hr_ref):
    h = jnp.dot(x_ref[...], w_ref[...],
                preferred_element_type=jnp.float32) + b_ref[...]
    h_ref[...] = h
    r = jnp.maximum(h, 0.0)
    hr_ref[0] = r[:, :DH]
    hr_ref[1] = r[:, DH:]


_proj = pl.pallas_call(
    _proj_body,
    grid=(NB,),
    in_specs=[
        pl.BlockSpec((BLK, D), lambda i: (i, 0)),
        pl.BlockSpec((D, D), lambda i: (0, 0)),
        pl.BlockSpec((1, D), lambda i: (0, 0)),
    ],
    out_specs=[
        pl.BlockSpec((BLK, D), lambda i: (i, 0)),
        pl.BlockSpec((NC, BLK, DH), lambda i: (0, i, 0)),
    ],
    out_shape=[
        jax.ShapeDtypeStruct((N, D), jnp.float32),
        jax.ShapeDtypeStruct((NC, N, DH), jnp.float32),
    ],
)


def _layer_body(eps_ref, h_ref, agg_ref, w1_ref, b1_ref, g_ref, be_ref,
                w2_ref, b2_ref, ho_ref, hro_ref):
    h = h_ref[...]
    agg = jnp.concatenate([agg_ref[0], agg_ref[1]], axis=-1)
    z = (1.0 + eps_ref[...]) * h + agg
    z = jnp.dot(z, w1_ref[...], preferred_element_type=jnp.float32) + b1_ref[...]
    mu = jnp.mean(z, axis=-1, keepdims=True)
    zc = z - mu
    var = jnp.mean(zc * zc, axis=-1, keepdims=True)
    z = zc * lax.rsqrt(var + 1e-5) * g_ref[...] + be_ref[...]
    z = jnp.maximum(z, 0.0)
    z = jnp.dot(z, w2_ref[...], preferred_element_type=jnp.float32) + b2_ref[...]
    ho = h + z
    ho_ref[...] = ho
    r = jnp.maximum(ho, 0.0)
    hro_ref[0] = r[:, :DH]
    hro_ref[1] = r[:, DH:]


_layer = pl.pallas_call(
    _layer_body,
    grid=(NB,),
    in_specs=[
        pl.BlockSpec((1, 1), lambda i: (0, 0)),
        pl.BlockSpec((BLK, D), lambda i: (i, 0)),
        pl.BlockSpec((NC, BLK, DH), lambda i: (0, i, 0)),
        pl.BlockSpec((D, 2 * D), lambda i: (0, 0)),
        pl.BlockSpec((1, 2 * D), lambda i: (0, 0)),
        pl.BlockSpec((1, 2 * D), lambda i: (0, 0)),
        pl.BlockSpec((1, 2 * D), lambda i: (0, 0)),
        pl.BlockSpec((2 * D, D), lambda i: (0, 0)),
        pl.BlockSpec((1, D), lambda i: (0, 0)),
    ],
    out_specs=[
        pl.BlockSpec((BLK, D), lambda i: (i, 0)),
        pl.BlockSpec((NC, BLK, DH), lambda i: (0, i, 0)),
    ],
    out_shape=[
        jax.ShapeDtypeStruct((N, D), jnp.float32),
        jax.ShapeDtypeStruct((NC, N, DH), jnp.float32),
    ],
)


def _pool_body(b3_ref, h_ref, wo1_ref, bo1_ref, wo2_ref, bo2_ref,
               out_ref, acc_ref):
    i = pl.program_id(0)

    @pl.when(i == 0)
    def _():
        acc_ref[...] = jnp.zeros_like(acc_ref)

    b = b3_ref[0, 0, :]
    onehot = (b[None, :] == lax.broadcasted_iota(jnp.int32, (G, BLK), 0)
              ).astype(jnp.float32)
    acc_ref[...] += jnp.dot(onehot, h_ref[...],
                            preferred_element_type=jnp.float32)

    @pl.when(i == NB - 1)
    def _():
        p = acc_ref[...]
        t = jnp.maximum(
            jnp.dot(p, wo1_ref[...], preferred_element_type=jnp.float32)
            + bo1_ref[...], 0.0)
        o = jnp.dot(t, wo2_ref[...], preferred_element_type=jnp.float32) \
            + bo2_ref[...]
        mask = (lax.broadcasted_iota(jnp.int32, (G, D), 1) == 0
                ).astype(jnp.float32)
        out_ref[...] = o * mask


_pool = pl.pallas_call(
    _pool_body,
    grid=(NB,),
    in_specs=[
        pl.BlockSpec((1, 1, BLK), lambda i: (i, 0, 0)),
        pl.BlockSpec((BLK, D), lambda i: (i, 0)),
        pl.BlockSpec((D, 2 * D), lambda i: (0, 0)),
        pl.BlockSpec((1, 2 * D), lambda i: (0, 0)),
        pl.BlockSpec((2 * D, D), lambda i: (0, 0)),
        pl.BlockSpec((1, 1), lambda i: (0, 0)),
    ],
    out_specs=pl.BlockSpec((G, D), lambda i: (0, 0)),
    out_shape=jax.ShapeDtypeStruct((G, D), jnp.float32),
    scratch_shapes=[pltpu.VMEM((G, D), jnp.float32)],
)


@jax.jit
def _run(x, edge_index, batch, params):
    src = edge_index[0].reshape(NS, NCHUNK, CH)
    dst = edge_index[1].reshape(NS, NCHUNK, CH)
    batch3 = batch.reshape(NB, 1, BLK)

    h, hr = _proj(x, params['W_in'], params['b_in'].reshape(1, D))
    for lp in params['layers']:
        agg2 = _sc_aggregate(hr, src, dst)
        h, hr = _layer(lp['eps'].reshape(1, 1), h, agg2,
                       lp['W1'], lp['b1'].reshape(1, -1),
                       lp['g'].reshape(1, -1), lp['be'].reshape(1, -1),
                       lp['W2'], lp['b2'].reshape(1, -1))
    wo2p = jnp.pad(params['Wo2'], ((0, 0), (0, D - 1)))
    outp = _pool(batch3, h, params['Wo1'], params['bo1'].reshape(1, -1),
                 wo2p, params['bo2'].reshape(1, 1))
    return outp[:, 0]


def kernel(x, edge_index, batch, params):
    return _run(x, edge_index, batch, params)


# speedup vs baseline: 10.7992x; 1.2846x over previous
import functools

import jax
import jax.numpy as jnp
from jax import lax
from jax.experimental import pallas as pl
from jax.experimental.pallas import tpu as pltpu
from jax.experimental.pallas import tpu_sc as plsc

N = 10000
E = 320000
D = 128
G = 16

NC = 2
NS = 16
DH = D // NC
EPS_ = E // NS
CH = 80
NCHUNK = EPS_ // CH
NBUF = 5
RPS = N // NS
ZR = 125

BLK = 1000
NB = N // BLK


def _sc_body(hr, srcr, dstr, out, sidx, didx, rows, zbuf, agg, gsem, ssem):
    cid = lax.axis_index("c")
    sid = lax.axis_index("s")

    pltpu.sync_copy(srcr.at[sid], sidx)
    pltpu.sync_copy(dstr.at[sid], didx)

    zv = jnp.zeros((16,), jnp.float32)

    def zrow(i, c):
        def zcol(k, c2):
            zbuf[i, pl.ds(k * 16, 16)] = zv
            return c2
        return lax.fori_loop(0, DH // 16, zcol, c)

    lax.fori_loop(0, ZR, zrow, 0)
    for r in range(RPS // ZR):
        pltpu.sync_copy(zbuf, agg.at[pl.ds(sid * RPS + r * ZR, ZR)])
    plsc.subcore_barrier()

    table = hr.at[cid]
    for b in range(NBUF):
        pltpu.async_copy(table.at[sidx.at[b]], rows.at[b], gsem.at[b])

    def group(g, c):
        j0 = g * NBUF
        for b in range(NBUF):
            j = j0 + b
            pltpu.make_async_copy(table.at[sidx.at[j]], rows.at[b],
                                  gsem.at[b]).wait()
            pltpu.async_copy(rows.at[b], agg.at[didx.at[j]], ssem.at[b],
                             add=True)
        for b in range(NBUF):
            j = j0 + b
            jn = j + NBUF
            pltpu.make_async_copy(rows.at[b], agg.at[didx.at[j]],
                                  ssem.at[b]).wait()

            @pl.when(jn < NCHUNK)
            def _():
                pltpu.async_copy(table.at[sidx.at[jn]], rows.at[b], gsem.at[b])
        return c

    lax.fori_loop(0, NCHUNK // NBUF, group, 0)
    plsc.subcore_barrier()

    start = pl.multiple_of(sid * 624, 16)

    @pl.when(sid < NS - 1)
    def _():
        pltpu.sync_copy(agg.at[pl.ds(start, 624)],
                        out.at[cid, pl.ds(start, 624)])

    @pl.when(sid == NS - 1)
    def _():
        pltpu.sync_copy(agg.at[pl.ds(start, 640)],
                        out.at[cid, pl.ds(start, 640)])


_sc_aggregate = pl.kernel(
    _sc_body,
    out_type=jax.ShapeDtypeStruct((NC, N, DH), jnp.float32),
    mesh=plsc.VectorSubcoreMesh(core_axis_name="c", subcore_axis_name="s"),
    scratch_types=[
        pltpu.VMEM((NCHUNK, CH), jnp.int32),
        pltpu.VMEM((NCHUNK, CH), jnp.int32),
        pltpu.VMEM((NBUF, CH, DH), jnp.float32),
        pltpu.VMEM((ZR, DH), jnp.float32),
        pltpu.VMEM_SHARED((N, DH), jnp.float32),
        pltpu.SemaphoreType.DMA((NBUF,)),
        pltpu.SemaphoreType.DMA((NBUF,)),
    ],
    compiler_params=pltpu.CompilerParams(use_tc_tiling_on_sc=False),
)


def _proj_body(x_ref, w_ref, b_ref, h_ref, hr_ref):
    h = jnp.dot(x_ref[...], w_ref[...],
                preferred_element_type=jnp.float32) + b_ref[...]
    h_ref[...] = h
    r = jnp.maximum(h, 0.0)
    hr_ref[0] = r[:, :DH]
    hr_ref[1] = r[:, DH:]


_proj = pl.pallas_call(
    _proj_body,
    grid=(NB,),
    in_specs=[
        pl.BlockSpec((BLK, D), lambda i: (i, 0)),
        pl.BlockSpec((D, D), lambda i: (0, 0)),
        pl.BlockSpec((1, D), lambda i: (0, 0)),
    ],
    out_specs=[
        pl.BlockSpec((BLK, D), lambda i: (i, 0)),
        pl.BlockSpec((NC, BLK, DH), lambda i: (0, i, 0)),
    ],
    out_shape=[
        jax.ShapeDtypeStruct((N, D), jnp.float32),
        jax.ShapeDtypeStruct((NC, N, DH), jnp.float32),
    ],
)


def _layer_body(eps_ref, h_ref, agg_ref, w1_ref, b1_ref, g_ref, be_ref,
                w2_ref, b2_ref, ho_ref, hro_ref):
    h = h_ref[...]
    agg = jnp.concatenate([agg_ref[0], agg_ref[1]], axis=-1)
    z = (1.0 + eps_ref[...]) * h + agg
    z = jnp.dot(z, w1_ref[...], preferred_element_type=jnp.float32) + b1_ref[...]
    mu = jnp.mean(z, axis=-1, keepdims=True)
    zc = z - mu
    var = jnp.mean(zc * zc, axis=-1, keepdims=True)
    z = zc * lax.rsqrt(var + 1e-5) * g_ref[...] + be_ref[...]
    z = jnp.maximum(z, 0.0)
    z = jnp.dot(z, w2_ref[...], preferred_element_type=jnp.float32) + b2_ref[...]
    ho = h + z
    ho_ref[...] = ho
    r = jnp.maximum(ho, 0.0)
    hro_ref[0] = r[:, :DH]
    hro_ref[1] = r[:, DH:]


_layer = pl.pallas_call(
    _layer_body,
    grid=(NB,),
    in_specs=[
        pl.BlockSpec((1, 1), lambda i: (0, 0)),
        pl.BlockSpec((BLK, D), lambda i: (i, 0)),
        pl.BlockSpec((NC, BLK, DH), lambda i: (0, i, 0)),
        pl.BlockSpec((D, 2 * D), lambda i: (0, 0)),
        pl.BlockSpec((1, 2 * D), lambda i: (0, 0)),
        pl.BlockSpec((1, 2 * D), lambda i: (0, 0)),
        pl.BlockSpec((1, 2 * D), lambda i: (0, 0)),
        pl.BlockSpec((2 * D, D), lambda i: (0, 0)),
        pl.BlockSpec((1, D), lambda i: (0, 0)),
    ],
    out_specs=[
        pl.BlockSpec((BLK, D), lambda i: (i, 0)),
        pl.BlockSpec((NC, BLK, DH), lambda i: (0, i, 0)),
    ],
    out_shape=[
        jax.ShapeDtypeStruct((N, D), jnp.float32),
        jax.ShapeDtypeStruct((NC, N, DH), jnp.float32),
    ],
)


def _last_body(eps_ref, h_ref, agg_ref, w1_ref, b1_ref, g_ref, be_ref,
               w2_ref, b2_ref, b3_ref, wo1_ref, bo1_ref, wo2_ref, bo2_ref,
               out_ref, acc_ref):
    i = pl.program_id(0)
    h = h_ref[...]
    agg = jnp.concatenate([agg_ref[0], agg_ref[1]], axis=-1)
    z = (1.0 + eps_ref[...]) * h + agg
    z = jnp.dot(z, w1_ref[...], preferred_element_type=jnp.float32) + b1_ref[...]
    mu = jnp.mean(z, axis=-1, keepdims=True)
    zc = z - mu
    var = jnp.mean(zc * zc, axis=-1, keepdims=True)
    z = zc * lax.rsqrt(var + 1e-5) * g_ref[...] + be_ref[...]
    z = jnp.maximum(z, 0.0)
    z = jnp.dot(z, w2_ref[...], preferred_element_type=jnp.float32) + b2_ref[...]
    ho = h + z

    @pl.when(i == 0)
    def _():
        acc_ref[...] = jnp.zeros_like(acc_ref)

    b = b3_ref[0, 0, :]
    onehot = (b[None, :] == lax.broadcasted_iota(jnp.int32, (G, BLK), 0)
              ).astype(jnp.float32)
    acc_ref[...] += jnp.dot(onehot, ho, preferred_element_type=jnp.float32)

    @pl.when(i == NB - 1)
    def _():
        p = acc_ref[...]
        t = jnp.maximum(
            jnp.dot(p, wo1_ref[...], preferred_element_type=jnp.float32)
            + bo1_ref[...], 0.0)
        o = jnp.dot(t, wo2_ref[...], preferred_element_type=jnp.float32) \
            + bo2_ref[...]
        mask = (lax.broadcasted_iota(jnp.int32, (G, D), 1) == 0
                ).astype(jnp.float32)
        out_ref[...] = o * mask


_last = pl.pallas_call(
    _last_body,
    grid=(NB,),
    in_specs=[
        pl.BlockSpec((1, 1), lambda i: (0, 0)),
        pl.BlockSpec((BLK, D), lambda i: (i, 0)),
        pl.BlockSpec((NC, BLK, DH), lambda i: (0, i, 0)),
        pl.BlockSpec((D, 2 * D), lambda i: (0, 0)),
        pl.BlockSpec((1, 2 * D), lambda i: (0, 0)),
        pl.BlockSpec((1, 2 * D), lambda i: (0, 0)),
        pl.BlockSpec((1, 2 * D), lambda i: (0, 0)),
        pl.BlockSpec((2 * D, D), lambda i: (0, 0)),
        pl.BlockSpec((1, D), lambda i: (0, 0)),
        pl.BlockSpec((1, 1, BLK), lambda i: (i, 0, 0)),
        pl.BlockSpec((D, 2 * D), lambda i: (0, 0)),
        pl.BlockSpec((1, 2 * D), lambda i: (0, 0)),
        pl.BlockSpec((2 * D, D), lambda i: (0, 0)),
        pl.BlockSpec((1, 1), lambda i: (0, 0)),
    ],
    out_specs=pl.BlockSpec((G, D), lambda i: (0, 0)),
    out_shape=jax.ShapeDtypeStruct((G, D), jnp.float32),
    scratch_shapes=[pltpu.VMEM((G, D), jnp.float32)],
)


@jax.jit
def _run(x, edge_index, batch, params):
    src = edge_index[0].reshape(NS, NCHUNK, CH)
    dst = edge_index[1].reshape(NS, NCHUNK, CH)
    batch3 = batch.reshape(NB, 1, BLK)

    h, hr = _proj(x, params['W_in'], params['b_in'].reshape(1, D))
    for lp in params['layers'][:-1]:
        agg2 = _sc_aggregate(hr, src, dst)
        h, hr = _layer(lp['eps'].reshape(1, 1), h, agg2,
                       lp['W1'], lp['b1'].reshape(1, -1),
                       lp['g'].reshape(1, -1), lp['be'].reshape(1, -1),
                       lp['W2'], lp['b2'].reshape(1, -1))
    lp = params['layers'][-1]
    agg2 = _sc_aggregate(hr, src, dst)
    wo2p = jnp.pad(params['Wo2'], ((0, 0), (0, D - 1)))
    outp = _last(lp['eps'].reshape(1, 1), h, agg2,
                 lp['W1'], lp['b1'].reshape(1, -1),
                 lp['g'].reshape(1, -1), lp['be'].reshape(1, -1),
                 lp['W2'], lp['b2'].reshape(1, -1),
                 batch3, params['Wo1'], params['bo1'].reshape(1, -1),
                 wo2p, params['bo2'].reshape(1, 1))
    return outp[:, 0]


def kernel(x, edge_index, batch, params):
    return _run(x, edge_index, batch, params)


# speedup vs baseline: 11.0666x; 1.0248x over previous
import functools

import jax
import jax.numpy as jnp
from jax import lax
from jax.experimental import pallas as pl
from jax.experimental.pallas import tpu as pltpu
from jax.experimental.pallas import tpu_sc as plsc

N = 10000
E = 320000
D = 128
G = 16

NC = 2
NS = 16
DH = D // NC
EPS_ = E // NS
CH = 125
NCHUNK = EPS_ // CH
NBUF = 5
RPS = N // NS
ZR = 125

BLK = 1000
NB = N // BLK


def _sc_body(hr, srcr, dstr, out, sidx, didx, rows, zbuf, agg, gsem, ssem):
    cid = lax.axis_index("c")
    sid = lax.axis_index("s")

    icp0 = pltpu.async_copy(srcr.at[sid], sidx, gsem.at[0])
    icp1 = pltpu.async_copy(dstr.at[sid], didx, gsem.at[1])

    zv = jnp.zeros((16,), jnp.float32)

    def zrow(i, c):
        def zcol(k, c2):
            zbuf[i, pl.ds(k * 16, 16)] = zv
            return c2
        return lax.fori_loop(0, DH // 16, zcol, c)

    lax.fori_loop(0, ZR, zrow, 0)
    icp0.wait()
    icp1.wait()

    table = hr.at[cid]
    for b in range(NBUF):
        pltpu.async_copy(table.at[sidx.at[b]], rows.at[b], gsem.at[b])
    for r in range(RPS // ZR):
        pltpu.sync_copy(zbuf, agg.at[pl.ds(sid * RPS + r * ZR, ZR)])
    plsc.subcore_barrier()


    def group(g, c):
        j0 = g * NBUF
        for b in range(NBUF):
            j = j0 + b
            pltpu.make_async_copy(table.at[sidx.at[j]], rows.at[b],
                                  gsem.at[b]).wait()
            pltpu.async_copy(rows.at[b], agg.at[didx.at[j]], ssem.at[b],
                             add=True)
        for b in range(NBUF):
            j = j0 + b
            jn = j + NBUF
            pltpu.make_async_copy(rows.at[b], agg.at[didx.at[j]],
                                  ssem.at[b]).wait()

            @pl.when(jn < NCHUNK)
            def _():
                pltpu.async_copy(table.at[sidx.at[jn]], rows.at[b], gsem.at[b])
        return c

    lax.fori_loop(0, NCHUNK // NBUF, group, 0)
    plsc.subcore_barrier()

    start = pl.multiple_of(sid * 624, 16)

    @pl.when(sid < NS - 1)
    def _():
        pltpu.sync_copy(agg.at[pl.ds(start, 624)],
                        out.at[cid, pl.ds(start, 624)])

    @pl.when(sid == NS - 1)
    def _():
        pltpu.sync_copy(agg.at[pl.ds(start, 640)],
                        out.at[cid, pl.ds(start, 640)])


_sc_aggregate = pl.kernel(
    _sc_body,
    out_type=jax.ShapeDtypeStruct((NC, N, DH), jnp.float32),
    mesh=plsc.VectorSubcoreMesh(core_axis_name="c", subcore_axis_name="s"),
    scratch_types=[
        pltpu.VMEM((NCHUNK, CH), jnp.int32),
        pltpu.VMEM((NCHUNK, CH), jnp.int32),
        pltpu.VMEM((NBUF, CH, DH), jnp.float32),
        pltpu.VMEM((ZR, DH), jnp.float32),
        pltpu.VMEM_SHARED((N, DH), jnp.float32),
        pltpu.SemaphoreType.DMA((NBUF,)),
        pltpu.SemaphoreType.DMA((NBUF,)),
    ],
    compiler_params=pltpu.CompilerParams(use_tc_tiling_on_sc=False),
)


def _proj_body(x_ref, w_ref, b_ref, h_ref, hr_ref):
    h = jnp.dot(x_ref[...], w_ref[...],
                preferred_element_type=jnp.float32) + b_ref[...]
    h_ref[...] = h
    r = jnp.maximum(h, 0.0)
    hr_ref[0] = r[:, :DH]
    hr_ref[1] = r[:, DH:]


_proj = pl.pallas_call(
    _proj_body,
    grid=(NB,),
    in_specs=[
        pl.BlockSpec((BLK, D), lambda i: (i, 0)),
        pl.BlockSpec((D, D), lambda i: (0, 0)),
        pl.BlockSpec((1, D), lambda i: (0, 0)),
    ],
    out_specs=[
        pl.BlockSpec((BLK, D), lambda i: (i, 0)),
        pl.BlockSpec((NC, BLK, DH), lambda i: (0, i, 0)),
    ],
    out_shape=[
        jax.ShapeDtypeStruct((N, D), jnp.float32),
        jax.ShapeDtypeStruct((NC, N, DH), jnp.float32),
    ],
)


def _layer_body(eps_ref, h_ref, agg_ref, w1_ref, b1_ref, g_ref, be_ref,
                w2_ref, b2_ref, ho_ref, hro_ref):
    h = h_ref[...]
    agg = jnp.concatenate([agg_ref[0], agg_ref[1]], axis=-1)
    z = (1.0 + eps_ref[...]) * h + agg
    z = jnp.dot(z, w1_ref[...], preferred_element_type=jnp.float32) + b1_ref[...]
    mu = jnp.mean(z, axis=-1, keepdims=True)
    zc = z - mu
    var = jnp.mean(zc * zc, axis=-1, keepdims=True)
    z = zc * lax.rsqrt(var + 1e-5) * g_ref[...] + be_ref[...]
    z = jnp.maximum(z, 0.0)
    z = jnp.dot(z, w2_ref[...], preferred_element_type=jnp.float32) + b2_ref[...]
    ho = h + z
    ho_ref[...] = ho
    r = jnp.maximum(ho, 0.0)
    hro_ref[0] = r[:, :DH]
    hro_ref[1] = r[:, DH:]


_layer = pl.pallas_call(
    _layer_body,
    grid=(NB,),
    in_specs=[
        pl.BlockSpec((1, 1), lambda i: (0, 0)),
        pl.BlockSpec((BLK, D), lambda i: (i, 0)),
        pl.BlockSpec((NC, BLK, DH), lambda i: (0, i, 0)),
        pl.BlockSpec((D, 2 * D), lambda i: (0, 0)),
        pl.BlockSpec((1, 2 * D), lambda i: (0, 0)),
        pl.BlockSpec((1, 2 * D), lambda i: (0, 0)),
        pl.BlockSpec((1, 2 * D), lambda i: (0, 0)),
        pl.BlockSpec((2 * D, D), lambda i: (0, 0)),
        pl.BlockSpec((1, D), lambda i: (0, 0)),
    ],
    out_specs=[
        pl.BlockSpec((BLK, D), lambda i: (i, 0)),
        pl.BlockSpec((NC, BLK, DH), lambda i: (0, i, 0)),
    ],
    out_shape=[
        jax.ShapeDtypeStruct((N, D), jnp.float32),
        jax.ShapeDtypeStruct((NC, N, DH), jnp.float32),
    ],
)


def _last_body(eps_ref, h_ref, agg_ref, w1_ref, b1_ref, g_ref, be_ref,
               w2_ref, b2_ref, b3_ref, wo1_ref, bo1_ref, wo2_ref, bo2_ref,
               out_ref, acc_ref):
    i = pl.program_id(0)
    h = h_ref[...]
    agg = jnp.concatenate([agg_ref[0], agg_ref[1]], axis=-1)
    z = (1.0 + eps_ref[...]) * h + agg
    z = jnp.dot(z, w1_ref[...], preferred_element_type=jnp.float32) + b1_ref[...]
    mu = jnp.mean(z, axis=-1, keepdims=True)
    zc = z - mu
    var = jnp.mean(zc * zc, axis=-1, keepdims=True)
    z = zc * lax.rsqrt(var + 1e-5) * g_ref[...] + be_ref[...]
    z = jnp.maximum(z, 0.0)
    z = jnp.dot(z, w2_ref[...], preferred_element_type=jnp.float32) + b2_ref[...]
    ho = h + z

    @pl.when(i == 0)
    def _():
        acc_ref[...] = jnp.zeros_like(acc_ref)

    b = b3_ref[0, 0, :]
    onehot = (b[None, :] == lax.broadcasted_iota(jnp.int32, (G, BLK), 0)
              ).astype(jnp.float32)
    acc_ref[...] += jnp.dot(onehot, ho, preferred_element_type=jnp.float32)

    @pl.when(i == NB - 1)
    def _():
        p = acc_ref[...]
        t = jnp.maximum(
            jnp.dot(p, wo1_ref[...], preferred_element_type=jnp.float32)
            + bo1_ref[...], 0.0)
        o = jnp.dot(t, wo2_ref[...], preferred_element_type=jnp.float32) \
            + bo2_ref[...]
        mask = (lax.broadcasted_iota(jnp.int32, (G, D), 1) == 0
                ).astype(jnp.float32)
        out_ref[...] = o * mask


_last = pl.pallas_call(
    _last_body,
    grid=(NB,),
    in_specs=[
        pl.BlockSpec((1, 1), lambda i: (0, 0)),
        pl.BlockSpec((BLK, D), lambda i: (i, 0)),
        pl.BlockSpec((NC, BLK, DH), lambda i: (0, i, 0)),
        pl.BlockSpec((D, 2 * D), lambda i: (0, 0)),
        pl.BlockSpec((1, 2 * D), lambda i: (0, 0)),
        pl.BlockSpec((1, 2 * D), lambda i: (0, 0)),
        pl.BlockSpec((1, 2 * D), lambda i: (0, 0)),
        pl.BlockSpec((2 * D, D), lambda i: (0, 0)),
        pl.BlockSpec((1, D), lambda i: (0, 0)),
        pl.BlockSpec((1, 1, BLK), lambda i: (i, 0, 0)),
        pl.BlockSpec((D, 2 * D), lambda i: (0, 0)),
        pl.BlockSpec((1, 2 * D), lambda i: (0, 0)),
        pl.BlockSpec((2 * D, D), lambda i: (0, 0)),
        pl.BlockSpec((1, 1), lambda i: (0, 0)),
    ],
    out_specs=pl.BlockSpec((G, D), lambda i: (0, 0)),
    out_shape=jax.ShapeDtypeStruct((G, D), jnp.float32),
    scratch_shapes=[pltpu.VMEM((G, D), jnp.float32)],
)


@jax.jit
def _run(x, edge_index, batch, params):
    src = edge_index[0].reshape(NS, NCHUNK, CH)
    dst = edge_index[1].reshape(NS, NCHUNK, CH)
    batch3 = batch.reshape(NB, 1, BLK)

    h, hr = _proj(x, params['W_in'], params['b_in'].reshape(1, D))
    for lp in params['layers'][:-1]:
        agg2 = _sc_aggregate(hr, src, dst)
        h, hr = _layer(lp['eps'].reshape(1, 1), h, agg2,
                       lp['W1'], lp['b1'].reshape(1, -1),
                       lp['g'].reshape(1, -1), lp['be'].reshape(1, -1),
                       lp['W2'], lp['b2'].reshape(1, -1))
    lp = params['layers'][-1]
    agg2 = _sc_aggregate(hr, src, dst)
    wo2p = jnp.pad(params['Wo2'], ((0, 0), (0, D - 1)))
    outp = _last(lp['eps'].reshape(1, 1), h, agg2,
                 lp['W1'], lp['b1'].reshape(1, -1),
                 lp['g'].reshape(1, -1), lp['be'].reshape(1, -1),
                 lp['W2'], lp['b2'].reshape(1, -1),
                 batch3, params['Wo1'], params['bo1'].reshape(1, -1),
                 wo2p, params['bo2'].reshape(1, 1))
    return outp[:, 0]


def kernel(x, edge_index, batch, params):
    return _run(x, edge_index, batch, params)
